# baseline (speedup 1.0000x reference)
import functools

import jax
import jax.numpy as jnp
from jax import lax
from jax.experimental import pallas as pl
from jax.experimental.pallas import tpu as pltpu
from jax.experimental.pallas import tpu_sc as plsc

B, T, D = 16, 2048, 64
NC, NS, L = 2, 16, 16
NW = NC * NS
TPW = T // NW
DV = D // L

_mesh = plsc.VectorSubcoreMesh(core_axis_name="c", subcore_axis_name="s")


@functools.partial(
    pl.kernel,
    mesh=_mesh,
    compiler_params=pltpu.CompilerParams(use_tc_tiling_on_sc=False),
    out_type=jax.ShapeDtypeStruct((B, T, D), jnp.float32),
    scratch_types=[
        pltpu.VMEM((B, TPW), jnp.int32),
        pltpu.VMEM((B, TPW, D), jnp.float32),
        pltpu.VMEM((TPW, D), jnp.float32),
        pltpu.SemaphoreType.DMA,
    ],
)
def _emb_kernel(x_hbm, tok_hbm, pos_hbm, out_hbm, idx_v, rows_v, pos_v, sem):
    wid = lax.axis_index("s") * NC + lax.axis_index("c")
    p0 = wid * TPW

    for b in range(B):
        pltpu.sync_copy(x_hbm.at[b, pl.ds(p0, TPW)], idx_v.at[b])
    pltpu.sync_copy(pos_hbm.at[pl.ds(p0, TPW)], pos_v)

    copies = [
        pltpu.async_copy(tok_hbm.at[idx_v.at[b]], rows_v.at[b], sem)
        for b in range(B)
    ]
    for c in copies:
        c.wait()

    def add_pos(j, _):
        for c in range(DV):
            pv = pos_v[j, pl.ds(c * L, L)]
            for b in range(B):
                plsc.addupdate(rows_v.at[b, j, pl.ds(c * L, L)], pv)
        return _

    lax.fori_loop(0, TPW, add_pos, None)

    for b in range(B):
        pltpu.sync_copy(rows_v.at[b], out_hbm.at[b, pl.ds(p0, TPW)])


def kernel(x, token_table, pos_table):
    return _emb_kernel(x, token_table, pos_table)
